# baseline (speedup 1.0000x reference)
import functools

import jax
import jax.numpy as jnp
from jax import lax
from jax.experimental import pallas as pl
from jax.experimental.pallas import tpu as pltpu
from jax.experimental.pallas import tpu_sc as plsc

B = 1024
DIM = 256
NNZ = DIM * (DIM + 1) // 2
IN_PAD = DIM * DIM + 16
OUT_PAD = NNZ + 16


def kernel(input):
    xf = input.reshape(B, DIM * DIM)

    info = plsc.get_sparse_core_info()
    nc, ns = info.num_cores, info.num_subcores
    nw = nc * ns
    bpw = B // nw

    mesh = plsc.VectorSubcoreMesh(core_axis_name="c", subcore_axis_name="s")

    @functools.partial(
        pl.kernel,
        mesh=mesh,
        out_type=jax.ShapeDtypeStruct((B, NNZ), jnp.float32),
        scratch_types=[
            pltpu.VMEM((IN_PAD,), jnp.float32),
            pltpu.VMEM((OUT_PAD,), jnp.float32),
        ],
    )
    def tri_kernel(x_hbm, out_hbm, in_v, out_v):
        wid = lax.axis_index("s") * nc + lax.axis_index("c")

        def batch_body(bi, carry):
            b = wid * bpw + bi
            pltpu.sync_copy(x_hbm.at[b], in_v.at[pl.ds(0, DIM * DIM)])

            def row_body(i, carry):
                in_base = i * (DIM + 1)
                out_base = i * DIM - (i * (i - 1)) // 2
                nchunks = (DIM - i + 15) // 16

                def chunk_body(c, carry):
                    off = c * 16
                    out_v[pl.ds(out_base + off, 16)] = in_v[
                        pl.ds(in_base + off, 16)
                    ]
                    return carry

                return lax.fori_loop(0, nchunks, chunk_body, carry)

            lax.fori_loop(0, DIM, row_body, 0)
            pltpu.sync_copy(out_v.at[pl.ds(0, NNZ)], out_hbm.at[b])
            return carry

        lax.fori_loop(0, bpw, batch_body, 0)

    y = tri_kernel(xf)
    return y.reshape(B, NNZ, 1)


# speedup vs baseline: 1.7766x; 1.7766x over previous
import functools

import jax
import jax.numpy as jnp
import numpy as np
from jax import lax
from jax.experimental import pallas as pl
from jax.experimental.pallas import tpu as pltpu
from jax.experimental.pallas import tpu_sc as plsc

B = 1024
DIM = 256
NNZ = DIM * (DIM + 1) // 2
HALF = DIM * DIM // 2
LAST_FULL_ROW = DIM - 16
OUT_PAD = NNZ + 16


def _off(i):
    return DIM * i - (i * (i - 1)) // 2


def _chunks_for_row(i, base_row):
    n = DIM - i
    in_row = (i - base_row) * DIM + i
    out_row = _off(i)
    ch = [(in_row + 16 * c, out_row + 16 * c) for c in range(n // 16)]
    if n % 16:
        ch.append((in_row + n - 16, out_row + n - 16))
    return ch


_HALF0 = [c for i in range(0, 128) for c in _chunks_for_row(i, 0)]
_HALF1 = [c for i in range(128, LAST_FULL_ROW + 1) for c in _chunks_for_row(i, 128)]


_TAIL = [((i - 128) * DIM + i, _off(i)) for i in range(LAST_FULL_ROW + 1, DIM)]

_OUT0 = 24640
_OUT1 = NNZ - _OUT0


def kernel(input):
    xf = input.reshape(B * DIM * DIM)

    info = plsc.get_sparse_core_info()
    nc, ns = info.num_cores, info.num_subcores
    nw = nc * ns
    bpw = B // nw

    mesh = plsc.VectorSubcoreMesh(core_axis_name="c", subcore_axis_name="s")

    @functools.partial(
        pl.kernel,
        mesh=mesh,
        out_type=jax.ShapeDtypeStruct((B * NNZ,), jnp.float32),
        scratch_types=[
            pltpu.VMEM((HALF,), jnp.float32),
            pltpu.VMEM((HALF + 16,), jnp.float32),
            pltpu.VMEM((OUT_PAD,), jnp.float32),
        ],
    )
    def tri_kernel(x_hbm, out_hbm, in0_v, in1_v, out_v):
        wid = lax.axis_index("s") * nc + lax.axis_index("c")

        def batch_body(bi, carry):
            b = wid * bpw + bi
            in_base = b * (DIM * DIM)
            out_base = b * NNZ
            pltpu.sync_copy(x_hbm.at[pl.ds(in_base, HALF)], in0_v)
            for src, dst in _HALF0:
                out_v[pl.ds(dst, 16)] = in0_v[pl.ds(src, 16)]
            pltpu.sync_copy(out_v.at[pl.ds(0, _OUT0)],
                            out_hbm.at[pl.ds(out_base, _OUT0)])
            pltpu.sync_copy(x_hbm.at[pl.ds(in_base + HALF, HALF)],
                            in1_v.at[pl.ds(0, HALF)])
            for src, dst in _HALF1:
                out_v[pl.ds(dst, 16)] = in1_v[pl.ds(src, 16)]
            for src_off, dst_off in _TAIL:
                out_v[pl.ds(dst_off, 16)] = in1_v[pl.ds(src_off, 16)]
            pltpu.sync_copy(out_v.at[pl.ds(_OUT0, _OUT1)],
                            out_hbm.at[pl.ds(out_base + _OUT0, _OUT1)])
            return carry

        lax.fori_loop(0, bpw, batch_body, 0)

    y = tri_kernel(xf)
    return y.reshape(B, NNZ, 1)


# speedup vs baseline: 2.2001x; 1.2384x over previous
import functools

import jax
import jax.numpy as jnp
import numpy as np
from jax import lax
from jax.experimental import pallas as pl
from jax.experimental.pallas import tpu as pltpu
from jax.experimental.pallas import tpu_sc as plsc

B = 1024
DIM = 256
NNZ = DIM * (DIM + 1) // 2
SEG_W = 16
SEGS_PER_MAT = DIM * DIM // SEG_W
NSEG = sum(16 - (r // 16) for r in range(DIM))
IDX_ROWS = NSEG // 128


def _off(i):
    return DIM * i - (i * (i - 1)) // 2


def _build_tables():
    seg_idx = []
    pos = {}
    for r in range(DIM):
        for s in range(r // 16, 16):
            pos[(r, s)] = len(seg_idx)
            seg_idx.append(r * 16 + s)
    stores = []
    for r in range(DIM - 1, -1, -1):
        s0, d = r // 16, r % 16
        if d > 0:
            stores.append((pos[(r, s0)], _off(r) - d))
        for s in range(s0 + (1 if d > 0 else 0), 16):
            stores.append((pos[(r, s)], _off(r) + 16 * s - r))
    return np.array(seg_idx, np.int32).reshape(IDX_ROWS, 128), stores


_SEG_IDX, _STORES = _build_tables()


def kernel(input):
    x_tbl = input.reshape(B * SEGS_PER_MAT, SEG_W)
    seg_idx = jnp.asarray(_SEG_IDX)

    info = plsc.get_sparse_core_info()
    nc, ns = info.num_cores, info.num_subcores
    nw = nc * ns
    bpw = B // nw
    ngrp = bpw // 2

    mesh = plsc.VectorSubcoreMesh(core_axis_name="c", subcore_axis_name="s")

    @functools.partial(
        pl.kernel,
        mesh=mesh,
        out_type=jax.ShapeDtypeStruct((B * NNZ,), jnp.float32),
        scratch_types=[
            pltpu.VMEM((NSEG, SEG_W), jnp.float32),
            pltpu.VMEM((NSEG, SEG_W), jnp.float32),
            pltpu.VMEM((NNZ,), jnp.float32),
            pltpu.VMEM((IDX_ROWS, 128), jnp.int32),
            pltpu.SemaphoreType.DMA,
            pltpu.SemaphoreType.DMA,
            pltpu.SemaphoreType.DMA,
        ],
        compiler_params=pltpu.CompilerParams(use_tc_tiling_on_sc=False),
    )
    def tri_kernel(x_hbm, sidx_hbm, out_hbm, stage0_v, stage1_v, out_v,
                   idx_v, in0_sem, in1_sem, out_sem):
        wid = lax.axis_index("s") * nc + lax.axis_index("c")
        b0 = wid * bpw
        pltpu.sync_copy(sidx_hbm, idx_v)

        def fire_gathers(b, stage_v, sem):
            tbl_b = x_hbm.at[pl.ds(b * SEGS_PER_MAT, SEGS_PER_MAT)]
            for j in range(IDX_ROWS):
                pltpu.async_copy(
                    tbl_b.at[idx_v.at[j]],
                    stage_v.at[pl.ds(128 * j, 128)],
                    sem,
                )

        def drain_gathers(stage_v, sem):
            pltpu.make_async_copy(
                x_hbm.at[pl.ds(0, NSEG)], stage_v, sem
            ).wait()

        def wait_out():
            pltpu.make_async_copy(
                out_v, out_hbm.at[pl.ds(b0 * NNZ, NNZ)], out_sem
            ).wait()

        def compact_and_flush(b, stage_v):
            for k, dst in _STORES:
                out_v[pl.ds(dst, 16)] = stage_v[k, :]
            pltpu.async_copy(out_v, out_hbm.at[pl.ds(b * NNZ, NNZ)], out_sem)

        fire_gathers(b0, stage0_v, in0_sem)

        def group_body(g, carry):
            be = b0 + 2 * g
            bo = be + 1
            bn = jnp.where(g + 1 < ngrp, be + 2, b0)

            fire_gathers(bo, stage1_v, in1_sem)
            drain_gathers(stage0_v, in0_sem)

            @pl.when(g > 0)
            def _():
                wait_out()

            compact_and_flush(be, stage0_v)

            fire_gathers(bn, stage0_v, in0_sem)
            drain_gathers(stage1_v, in1_sem)
            wait_out()
            compact_and_flush(bo, stage1_v)
            return carry

        lax.fori_loop(0, ngrp, group_body, 0)
        drain_gathers(stage0_v, in0_sem)
        wait_out()

    y = tri_kernel(x_tbl, seg_idx)
    return y.reshape(B, NNZ, 1)


# speedup vs baseline: 4.3215x; 1.9642x over previous
import functools

import jax
import jax.numpy as jnp
import numpy as np
from jax import lax
from jax.experimental import pallas as pl
from jax.experimental.pallas import tpu as pltpu
from jax.experimental.pallas import tpu_sc as plsc

B = 1024
DIM = 256
NNZ = DIM * (DIM + 1) // 2
SEG_W = 16
SEGS_PER_MAT = DIM * DIM // SEG_W
NSEG = sum(16 - (r // 16) for r in range(DIM))
IDX_ROWS = NSEG // 128


def _off(i):
    return DIM * i - (i * (i - 1)) // 2


def _build_tables():
    seg_idx = []
    pos = {}
    for r in range(DIM):
        for s in range(r // 16, 16):
            pos[(r, s)] = len(seg_idx)
            seg_idx.append((r // 8) * 128 + (s // 8) * 64 + (r % 8) * 8 + (s % 8))
    stores = []
    for r in range(DIM - 1, -1, -1):
        s0, d = r // 16, r % 16
        if d > 0:
            stores.append((pos[(r, s0)], _off(r) - d))
        for s in range(s0 + (1 if d > 0 else 0), 16):
            stores.append((pos[(r, s)], _off(r) + 16 * s - r))
    return np.array(seg_idx, np.int32).reshape(IDX_ROWS, 128), stores


_SEG_IDX, _STORES = _build_tables()


def kernel(input):
    x_tbl = (
        input.reshape(B, 32, 8, 2, 128)
        .transpose(0, 1, 3, 2, 4)
        .reshape(B * SEGS_PER_MAT, SEG_W)
    )
    seg_idx = jnp.asarray(_SEG_IDX)

    info = plsc.get_sparse_core_info()
    nc, ns = info.num_cores, info.num_subcores
    nw = nc * ns
    bpw = B // nw
    ngrp = bpw // 2

    mesh = plsc.VectorSubcoreMesh(core_axis_name="c", subcore_axis_name="s")

    @functools.partial(
        pl.kernel,
        mesh=mesh,
        out_type=jax.ShapeDtypeStruct((B * NNZ,), jnp.float32),
        scratch_types=[
            pltpu.VMEM((NSEG, SEG_W), jnp.float32),
            pltpu.VMEM((NSEG, SEG_W), jnp.float32),
            pltpu.VMEM((NNZ,), jnp.float32),
            pltpu.VMEM((IDX_ROWS, 128), jnp.int32),
            pltpu.SemaphoreType.DMA,
            pltpu.SemaphoreType.DMA,
            pltpu.SemaphoreType.DMA,
        ],
        compiler_params=pltpu.CompilerParams(use_tc_tiling_on_sc=False),
    )
    def tri_kernel(x_hbm, sidx_hbm, out_hbm, stage0_v, stage1_v, out_v,
                   idx_v, in0_sem, in1_sem, out_sem):
        wid = lax.axis_index("s") * nc + lax.axis_index("c")
        b0 = wid * bpw
        pltpu.sync_copy(sidx_hbm, idx_v)

        def fire_gathers(b, stage_v, sem):
            tbl_b = x_hbm.at[pl.ds(b * SEGS_PER_MAT, SEGS_PER_MAT)]
            for j in range(IDX_ROWS):
                pltpu.async_copy(
                    tbl_b.at[idx_v.at[j]],
                    stage_v.at[pl.ds(128 * j, 128)],
                    sem,
                )

        def drain_gathers(stage_v, sem):
            pltpu.make_async_copy(
                x_hbm.at[pl.ds(0, NSEG)], stage_v, sem
            ).wait()

        def wait_out():
            pltpu.make_async_copy(
                out_v, out_hbm.at[pl.ds(b0 * NNZ, NNZ)], out_sem
            ).wait()

        def compact_and_flush(b, stage_v):
            for k, dst in _STORES:
                out_v[pl.ds(dst, 16)] = stage_v[k, :]
            pltpu.async_copy(out_v, out_hbm.at[pl.ds(b * NNZ, NNZ)], out_sem)

        fire_gathers(b0, stage0_v, in0_sem)

        def group_body(g, carry):
            be = b0 + 2 * g
            bo = be + 1
            bn = jnp.where(g + 1 < ngrp, be + 2, b0)

            fire_gathers(bo, stage1_v, in1_sem)
            drain_gathers(stage0_v, in0_sem)

            @pl.when(g > 0)
            def _():
                wait_out()

            compact_and_flush(be, stage0_v)

            fire_gathers(bn, stage0_v, in0_sem)
            drain_gathers(stage1_v, in1_sem)
            wait_out()
            compact_and_flush(bo, stage1_v)
            return carry

        lax.fori_loop(0, ngrp, group_body, 0)
        drain_gathers(stage0_v, in0_sem)
        wait_out()

    y = tri_kernel(x_tbl, seg_idx)
    return y.reshape(B, NNZ, 1)
